# baseline (speedup 1.0000x reference)
import functools

import jax
import jax.numpy as jnp
from jax import lax
from jax.experimental import pallas as pl
from jax.experimental.pallas import tpu as pltpu
from jax.experimental.pallas import tpu_sc as plsc

N = 50000
D = 256
MAX_DEG = 10
N0 = 5000
ND = 4500

NC = 2
NS = 16
NW = NC * NS
CHUNK = 144
NDP = NW * CHUNK
PB = 5000
SUB = 72
NSUB = CHUNK // SUB
NCOLS = MAX_DEG * (MAX_DEG + 1) // 2

BLK = 1000
NBLK = N // BLK
HB = 500


def _sc_gather_sum(flat_idx, table):
  mesh = plsc.VectorSubcoreMesh(
      core_axis_name="c", subcore_axis_name="s", num_cores=NC, num_subcores=NS
  )

  @functools.partial(
      pl.kernel,
      out_type=jax.ShapeDtypeStruct((MAX_DEG * PB, D), jnp.float32),
      mesh=mesh,
      scratch_types=[
          pltpu.VMEM((NCOLS * NSUB, SUB), jnp.int32),
          pltpu.VMEM((CHUNK, D), jnp.float32),
          pltpu.VMEM((3 * SUB, D), jnp.float32),
          pltpu.SemaphoreType.DMA((5,)),
      ],
  )
  def run(idx_hbm, table_hbm, out_hbm, idx_v, acc_v, stage_v, sem):
    wid = lax.axis_index("s") * NC + lax.axis_index("c")
    start_w = wid * CHUNK
    pltpu.sync_copy(idx_hbm.at[wid], idx_v)

    def stage_wait(off, p):
      pltpu.make_async_copy(
          table_hbm.at[pl.ds(0, SUB)],
          stage_v.at[pl.ds(off, SUB)],
          sem.at[p],
      ).wait()

    rowbase = 0
    for d in range(1, MAX_DEG + 1):
      cp0 = pltpu.async_copy(
          table_hbm.at[idx_v.at[rowbase]],
          acc_v.at[pl.ds(0, SUB)],
          sem.at[3],
      )
      cp1 = pltpu.async_copy(
          table_hbm.at[idx_v.at[rowbase + 1]],
          acc_v.at[pl.ds(SUB, SUB)],
          sem.at[4],
      )
      nu = NSUB * (d - 1)
      if nu:
        for pr in range(min(2, nu)):
          pltpu.async_copy(
              table_hbm.at[idx_v.at[rowbase + NSUB + pr]],
              stage_v.at[pl.ds(pr * SUB, SUB)],
              sem.at[pr],
          )
      cp0.wait()
      cp1.wait()

      if nu:
        def body(u, carry):
          p = lax.rem(u, 3)
          off = p * SUB

          @pl.when(u + 2 < nu)
          def _():
            pn = lax.rem(u + 2, 3)
            pltpu.async_copy(
                table_hbm.at[idx_v.at[rowbase + NSUB + u + 2]],
                stage_v.at[pl.ds(pn * SUB, SUB)],
                sem.at[pn],
            )

          stage_wait(off, p)

          acc_off = lax.rem(u, 2) * SUB

          def add_row(r, c):
            for k in range(D // 16):
              plsc.addupdate(
                  acc_v.at[acc_off + r, pl.ds(k * 16, 16)],
                  stage_v[off + r, pl.ds(k * 16, 16)],
              )
            return c

          lax.fori_loop(0, SUB, add_row, 0)
          return carry

        lax.fori_loop(0, nu, body, 0)

      base = (d - 1) * PB + start_w
      pltpu.sync_copy(acc_v, out_hbm.at[pl.ds(base, CHUNK)])
      rowbase += NSUB * d

  return run(flat_idx, table)


def _tc_body(a_ref, s_lo_ref, s_hi_ref, ws_lo_ref, ws_hi_ref, wr_lo_ref,
             wr_hi_ref, b_lo_ref, b_hi_ref, o_ref):
  a = a_ref[0]
  o_ref[0, :HB] = (
      jnp.dot(a[:HB], ws_lo_ref[0], preferred_element_type=jnp.float32)
      + jnp.dot(s_lo_ref[0], wr_lo_ref[0], preferred_element_type=jnp.float32)
      + b_lo_ref[0]
  )
  o_ref[0, HB:] = (
      jnp.dot(a[HB:], ws_hi_ref[0], preferred_element_type=jnp.float32)
      + jnp.dot(s_hi_ref[0], wr_hi_ref[0], preferred_element_type=jnp.float32)
      + b_hi_ref[0]
  )


def _bucket(j):
  return jnp.where(j < 10, 0, (j - 10) // 9 + 1)


def _shalf(j):
  return jnp.where(j < 10, 0, (j - 10) + (j - 10) // 9)


_tc_matmul = pl.pallas_call(
    _tc_body,
    out_shape=jax.ShapeDtypeStruct((NBLK, BLK, D), jnp.float32),
    grid=(NBLK,),
    in_specs=[
        pl.BlockSpec((1, BLK, D), lambda i: (i, 0, 0)),
        pl.BlockSpec((1, HB, D), lambda i: (_shalf(2 * i), 0, 0)),
        pl.BlockSpec((1, HB, D), lambda i: (_shalf(2 * i + 1), 0, 0)),
        pl.BlockSpec((1, D, D), lambda i: (_bucket(2 * i), 0, 0)),
        pl.BlockSpec((1, D, D), lambda i: (_bucket(2 * i + 1), 0, 0)),
        pl.BlockSpec((1, D, D), lambda i: (_bucket(2 * i), 0, 0)),
        pl.BlockSpec((1, D, D), lambda i: (_bucket(2 * i + 1), 0, 0)),
        pl.BlockSpec((1, 1, D), lambda i: (_bucket(2 * i), 0, 0)),
        pl.BlockSpec((1, 1, D), lambda i: (_bucket(2 * i + 1), 0, 0)),
    ],
    out_specs=pl.BlockSpec((1, BLK, D), lambda i: (i, 0, 0)),
    compiler_params=pltpu.CompilerParams(
        dimension_semantics=("arbitrary",),
    ),
)


def kernel(atom_features, deg_slice, membership, deg_adj_1, deg_adj_2,
           deg_adj_3, deg_adj_4, deg_adj_5, deg_adj_6, deg_adj_7, deg_adj_8,
           deg_adj_9, deg_adj_10, W, b):
  adj = [deg_adj_1, deg_adj_2, deg_adj_3, deg_adj_4, deg_adj_5, deg_adj_6,
         deg_adj_7, deg_adj_8, deg_adj_9, deg_adj_10]
  allc = jnp.concatenate(
      [jnp.concatenate([a.T, a.T[:, : NDP - ND]], axis=1) for a in adj],
      axis=0,
  )
  flat_idx = (
      allc.reshape(NCOLS, NW, NSUB, SUB)
      .transpose(1, 0, 2, 3)
      .reshape(NW, NCOLS * NSUB, SUB)
  )

  summed = _sc_gather_sum(flat_idx, atom_features)

  w_self = jnp.concatenate([W[20:21], W[1:20:2]], axis=0)
  w_rel = jnp.concatenate(
      [jnp.zeros((1, D, D), W.dtype), W[0:20:2]], axis=0
  )
  b_comb = jnp.concatenate([b[20:21], b[0:20:2] + b[1:20:2]], axis=0)
  b_comb = b_comb.reshape(MAX_DEG + 1, 1, D)

  out = _tc_matmul(
      atom_features.reshape(NBLK, BLK, D),
      summed.reshape(2 * NBLK, HB, D),
      summed.reshape(2 * NBLK, HB, D),
      w_self,
      w_self,
      w_rel,
      w_rel,
      b_comb,
      b_comb,
  )
  return out.reshape(N, D)


# speedup vs baseline: 1.7226x; 1.7226x over previous
import functools

import jax
import jax.numpy as jnp
from jax import lax
from jax.experimental import pallas as pl
from jax.experimental.pallas import tpu as pltpu
from jax.experimental.pallas import tpu_sc as plsc

N = 50000
D = 256
MAX_DEG = 10
N0 = 5000
ND = 4500

NC = 2
NS = 16
NW = NC * NS
CHUNK = 144
NDP = NW * CHUNK
PB = 5000
SUB = 72
NSUB = CHUNK // SUB
NCOLS = MAX_DEG * (MAX_DEG + 1) // 2

BLK = 1000
NBLK = N // BLK
HB = 500


def _sc_gather_sum(flat_idx, table):
  mesh = plsc.VectorSubcoreMesh(
      core_axis_name="c", subcore_axis_name="s", num_cores=NC, num_subcores=NS
  )

  @functools.partial(
      pl.kernel,
      out_type=jax.ShapeDtypeStruct((MAX_DEG * PB, D), jnp.float32),
      mesh=mesh,
      scratch_types=[
          pltpu.VMEM((NCOLS * NSUB, SUB), jnp.int32),
          pltpu.VMEM((CHUNK, D), jnp.float32),
          pltpu.VMEM((CHUNK, D), jnp.float32),
          pltpu.SemaphoreType.DMA((4,)),
      ],
  )
  def run(idx_hbm, table_hbm, out_hbm, idx_v, acc_v, stage_v, sem):
    wid = lax.axis_index("s") * NC + lax.axis_index("c")
    start_w = wid * CHUNK
    pltpu.sync_copy(idx_hbm.at[wid], idx_v)

    def stage_wait(off, p):
      pltpu.make_async_copy(
          table_hbm.at[pl.ds(0, SUB)],
          stage_v.at[pl.ds(off, SUB)],
          sem.at[p],
      ).wait()

    rowbase = 0
    for d in range(1, MAX_DEG + 1):
      cp0 = pltpu.async_copy(
          table_hbm.at[idx_v.at[rowbase]],
          acc_v.at[pl.ds(0, SUB)],
          sem.at[2],
      )
      cp1 = pltpu.async_copy(
          table_hbm.at[idx_v.at[rowbase + 1]],
          acc_v.at[pl.ds(SUB, SUB)],
          sem.at[3],
      )
      nu = NSUB * (d - 1)
      if nu:
        pltpu.async_copy(
            table_hbm.at[idx_v.at[rowbase + NSUB]],
            stage_v.at[pl.ds(0, SUB)],
            sem.at[0],
        )
      cp0.wait()
      cp1.wait()

      if nu:
        def body(u, carry):
          p = lax.rem(u, 2)
          off = p * SUB

          @pl.when(u + 1 < nu)
          def _():
            pn = lax.rem(u + 1, 2)
            pltpu.async_copy(
                table_hbm.at[idx_v.at[rowbase + NSUB + u + 1]],
                stage_v.at[pl.ds(pn * SUB, SUB)],
                sem.at[pn],
            )

          stage_wait(off, p)

          def add_row(r, c):
            row = off + r
            for k in range(D // 16):
              plsc.addupdate(
                  acc_v.at[row, pl.ds(k * 16, 16)],
                  stage_v[row, pl.ds(k * 16, 16)],
              )
            return c

          lax.fori_loop(0, SUB, add_row, 0)
          return carry

        lax.fori_loop(0, nu, body, 0)

      base = (d - 1) * PB + start_w
      pltpu.sync_copy(acc_v, out_hbm.at[pl.ds(base, CHUNK)])
      rowbase += NSUB * d

  return run(flat_idx, table)


def _tc_body(a_ref, s_lo_ref, s_hi_ref, ws_lo_ref, ws_hi_ref, wr_lo_ref,
             wr_hi_ref, b_lo_ref, b_hi_ref, o_ref):
  a = a_ref[0]
  o_ref[0, :HB] = (
      jnp.dot(a[:HB], ws_lo_ref[0], preferred_element_type=jnp.float32)
      + jnp.dot(s_lo_ref[0], wr_lo_ref[0], preferred_element_type=jnp.float32)
      + b_lo_ref[0]
  )
  o_ref[0, HB:] = (
      jnp.dot(a[HB:], ws_hi_ref[0], preferred_element_type=jnp.float32)
      + jnp.dot(s_hi_ref[0], wr_hi_ref[0], preferred_element_type=jnp.float32)
      + b_hi_ref[0]
  )


def _bucket(j):
  return jnp.where(j < 10, 0, (j - 10) // 9 + 1)


def _shalf(j):
  return jnp.where(j < 10, 0, (j - 10) + (j - 10) // 9)


_tc_matmul = pl.pallas_call(
    _tc_body,
    out_shape=jax.ShapeDtypeStruct((NBLK, BLK, D), jnp.float32),
    grid=(NBLK,),
    in_specs=[
        pl.BlockSpec((1, BLK, D), lambda i: (i, 0, 0)),
        pl.BlockSpec((1, HB, D), lambda i: (_shalf(2 * i), 0, 0)),
        pl.BlockSpec((1, HB, D), lambda i: (_shalf(2 * i + 1), 0, 0)),
        pl.BlockSpec((1, D, D), lambda i: (_bucket(2 * i), 0, 0)),
        pl.BlockSpec((1, D, D), lambda i: (_bucket(2 * i + 1), 0, 0)),
        pl.BlockSpec((1, D, D), lambda i: (_bucket(2 * i), 0, 0)),
        pl.BlockSpec((1, D, D), lambda i: (_bucket(2 * i + 1), 0, 0)),
        pl.BlockSpec((1, 1, D), lambda i: (_bucket(2 * i), 0, 0)),
        pl.BlockSpec((1, 1, D), lambda i: (_bucket(2 * i + 1), 0, 0)),
    ],
    out_specs=pl.BlockSpec((1, BLK, D), lambda i: (i, 0, 0)),
    compiler_params=pltpu.CompilerParams(
        dimension_semantics=("arbitrary",),
    ),
)


def kernel(atom_features, deg_slice, membership, deg_adj_1, deg_adj_2,
           deg_adj_3, deg_adj_4, deg_adj_5, deg_adj_6, deg_adj_7, deg_adj_8,
           deg_adj_9, deg_adj_10, W, b):
  adj = [deg_adj_1, deg_adj_2, deg_adj_3, deg_adj_4, deg_adj_5, deg_adj_6,
         deg_adj_7, deg_adj_8, deg_adj_9, deg_adj_10]
  allc = jnp.concatenate(
      [jnp.concatenate([a.T, a.T[:, : NDP - ND]], axis=1) for a in adj],
      axis=0,
  )
  flat_idx = (
      allc.reshape(NCOLS, NW, NSUB, SUB)
      .transpose(1, 0, 2, 3)
      .reshape(NW, NCOLS * NSUB, SUB)
  )

  summed = _sc_gather_sum(flat_idx, atom_features)

  w_self = jnp.concatenate([W[20:21], W[1:20:2]], axis=0)
  w_rel = jnp.concatenate(
      [jnp.zeros((1, D, D), W.dtype), W[0:20:2]], axis=0
  )
  b_comb = jnp.concatenate([b[20:21], b[0:20:2] + b[1:20:2]], axis=0)
  b_comb = b_comb.reshape(MAX_DEG + 1, 1, D)

  out = _tc_matmul(
      atom_features.reshape(NBLK, BLK, D),
      summed.reshape(2 * NBLK, HB, D),
      summed.reshape(2 * NBLK, HB, D),
      w_self,
      w_self,
      w_rel,
      w_rel,
      b_comb,
      b_comb,
  )
  return out.reshape(N, D)
